# baseline (speedup 1.0000x reference)
import functools

import jax
import jax.numpy as jnp
from jax.experimental import pallas as pl
from jax.experimental.pallas import tpu as pltpu

F = 128


def _gcn_body(nf_ref, a_ref, wg_ref, bg_ref, wd1_ref, bd1_ref, wd2_ref,
              bd2_ref, wd3_ref, bd3_ref, out_ref, z_ref, h0_ref, h1_ref,
              pool_ref, *, br):
    l = pl.program_id(0)
    i = pl.program_id(1)
    nb = pl.num_programs(1)

    @pl.when(i == 0)
    def _():
        w = wg_ref[0]

        @pl.when(l == 0)
        def _():
            z_ref[...] = jnp.dot(nf_ref[...], w,
                                 preferred_element_type=jnp.float32)

        @pl.when(l % 2 == 1)
        def _():
            z_ref[...] = jnp.dot(h0_ref[...], w,
                                 preferred_element_type=jnp.float32)

        @pl.when((l > 0) & (l % 2 == 0))
        def _():
            z_ref[...] = jnp.dot(h1_ref[...], w,
                                 preferred_element_type=jnp.float32)

    acc = jnp.dot(a_ref[...], z_ref[...],
                  preferred_element_type=jnp.float32,
                  precision=jax.lax.Precision.HIGHEST)
    h_out = jnp.maximum(acc + bg_ref[0], 0.0)

    @pl.when(l % 2 == 0)
    def _():
        h0_ref[pl.ds(i * br, br), :] = h_out

    @pl.when(l % 2 == 1)
    def _():
        h1_ref[pl.ds(i * br, br), :] = h_out

    @pl.when(l == 4)
    def _():
        @pl.when(i == 0)
        def _():
            pool_ref[...] = jnp.zeros_like(pool_ref)

        pool_ref[...] += jnp.sum(h_out, axis=0, keepdims=True)

        @pl.when(i == nb - 1)
        def _():
            p = pool_ref[...]
            nrm = jnp.sqrt(jnp.sum(p * p))
            x = p / jnp.maximum(nrm, 1e-12)
            x = jnp.maximum(
                jnp.dot(x, wd1_ref[...],
                        preferred_element_type=jnp.float32) + bd1_ref[...],
                0.0)
            x = jnp.maximum(
                jnp.dot(x, wd2_ref[...],
                        preferred_element_type=jnp.float32) + bd2_ref[...],
                0.0)
            out_ref[...] = (jnp.dot(x, wd3_ref[...],
                                    preferred_element_type=jnp.float32)
                            + bd3_ref[...])


def _pad2(w, r, c):
    return jnp.pad(w, ((0, r - w.shape[0]), (0, c - w.shape[1])))


def kernel(node_feats, adj, Wg1, bg1, Wg2, bg2, Wg3, bg3, Wg4, bg4, Wg5, bg5,
           Wd1, bd1, Wd2, bd2, Wd3, bd3):
    n = adj.shape[0]
    br = 400 if n % 400 == 0 else n // 4
    nb = n // br

    nf_p = jnp.pad(node_feats, ((0, 0), (0, F - node_feats.shape[1])))
    wg = jnp.stack([_pad2(w, F, F) for w in (Wg1, Wg2, Wg3, Wg4, Wg5)])
    bg = jnp.stack([jnp.pad(b, (0, F - b.shape[0])).reshape(1, F)
                    for b in (bg1, bg2, bg3, bg4, bg5)])
    wd3_p = _pad2(Wd3, F, F)
    bd3_p = jnp.pad(bd3.reshape(1, 1), ((0, 0), (0, F - 1)))

    out = pl.pallas_call(
        functools.partial(_gcn_body, br=br),
        grid=(5, nb),
        in_specs=[
            pl.BlockSpec((n, F), lambda l, i: (0, 0)),
            pl.BlockSpec((br, n), lambda l, i: (i, 0)),
            pl.BlockSpec((1, F, F), lambda l, i: (l, 0, 0)),
            pl.BlockSpec((1, 1, F), lambda l, i: (l, 0, 0)),
            pl.BlockSpec((F, 256), lambda l, i: (0, 0)),
            pl.BlockSpec((1, 256), lambda l, i: (0, 0)),
            pl.BlockSpec((256, F), lambda l, i: (0, 0)),
            pl.BlockSpec((1, F), lambda l, i: (0, 0)),
            pl.BlockSpec((F, F), lambda l, i: (0, 0)),
            pl.BlockSpec((1, F), lambda l, i: (0, 0)),
        ],
        out_specs=pl.BlockSpec((1, F), lambda l, i: (0, 0)),
        out_shape=jax.ShapeDtypeStruct((1, F), jnp.float32),
        scratch_shapes=[
            pltpu.VMEM((n, F), jnp.float32),
            pltpu.VMEM((n, F), jnp.float32),
            pltpu.VMEM((n, F), jnp.float32),
            pltpu.VMEM((1, F), jnp.float32),
        ],
        compiler_params=pltpu.CompilerParams(
            dimension_semantics=("arbitrary", "arbitrary")),
    )(nf_p, adj, wg, bg.reshape(5, 1, F), Wd1, bd1.reshape(1, 256), Wd2,
      bd2.reshape(1, F), wd3_p, bd3_p)

    return out[0, :1]


# speedup vs baseline: 2.0300x; 2.0300x over previous
import functools

import jax
import jax.numpy as jnp
from jax.experimental import pallas as pl
from jax.experimental.pallas import tpu as pltpu

F = 128


def _gcn_body(nf_ref, a_ref, wg_ref, bg_ref, wd1_ref, bd1_ref, wd2_ref,
              bd2_ref, wd3_ref, bd3_ref, out_ref, zh_ref, zl_ref, h0_ref,
              h1_ref, pool_ref, *, br):
    l = pl.program_id(0)
    i = pl.program_id(1)
    nb = pl.num_programs(1)

    @pl.when(i == 0)
    def _():
        w = wg_ref[0]

        def _store_z(h):
            z = jnp.dot(h, w, preferred_element_type=jnp.float32,
                        precision=jax.lax.Precision.HIGHEST)
            zh = z.astype(jnp.bfloat16)
            zh_ref[...] = zh
            zl_ref[...] = (z - zh.astype(jnp.float32)).astype(jnp.bfloat16)

        @pl.when(l == 0)
        def _():
            _store_z(nf_ref[...])

        @pl.when(l % 2 == 1)
        def _():
            _store_z(h0_ref[...])

        @pl.when((l > 0) & (l % 2 == 0))
        def _():
            _store_z(h1_ref[...])

    a = a_ref[...]
    acc = (jnp.dot(a, zh_ref[...], preferred_element_type=jnp.float32)
           + jnp.dot(a, zl_ref[...], preferred_element_type=jnp.float32))
    h_out = jnp.maximum(acc + bg_ref[0], 0.0)

    @pl.when(l % 2 == 0)
    def _():
        h0_ref[pl.ds(i * br, br), :] = h_out

    @pl.when(l % 2 == 1)
    def _():
        h1_ref[pl.ds(i * br, br), :] = h_out

    @pl.when(l == 4)
    def _():
        @pl.when(i == 0)
        def _():
            pool_ref[...] = jnp.zeros_like(pool_ref)

        pool_ref[...] += jnp.sum(h_out, axis=0, keepdims=True)

        @pl.when(i == nb - 1)
        def _():
            p = pool_ref[...]
            nrm = jnp.sqrt(jnp.sum(p * p))
            x = p / jnp.maximum(nrm, 1e-12)
            x = jnp.maximum(
                jnp.dot(x, wd1_ref[...],
                        preferred_element_type=jnp.float32) + bd1_ref[...],
                0.0)
            x = jnp.maximum(
                jnp.dot(x, wd2_ref[...],
                        preferred_element_type=jnp.float32) + bd2_ref[...],
                0.0)
            out_ref[...] = (jnp.dot(x, wd3_ref[...],
                                    preferred_element_type=jnp.float32)
                            + bd3_ref[...])


def _pad2(w, r, c):
    return jnp.pad(w, ((0, r - w.shape[0]), (0, c - w.shape[1])))


def kernel(node_feats, adj, Wg1, bg1, Wg2, bg2, Wg3, bg3, Wg4, bg4, Wg5, bg5,
           Wd1, bd1, Wd2, bd2, Wd3, bd3):
    n = adj.shape[0]
    br = 400 if n % 400 == 0 else n // 4
    nb = n // br

    nf_p = jnp.pad(node_feats, ((0, 0), (0, F - node_feats.shape[1])))
    wg = jnp.stack([_pad2(w, F, F) for w in (Wg1, Wg2, Wg3, Wg4, Wg5)])
    bg = jnp.stack([jnp.pad(b, (0, F - b.shape[0])).reshape(1, F)
                    for b in (bg1, bg2, bg3, bg4, bg5)])
    wd3_p = _pad2(Wd3, F, F)
    bd3_p = jnp.pad(bd3.reshape(1, 1), ((0, 0), (0, F - 1)))

    adj_bf = adj.astype(jnp.bfloat16)

    out = pl.pallas_call(
        functools.partial(_gcn_body, br=br),
        grid=(5, nb),
        in_specs=[
            pl.BlockSpec((n, F), lambda l, i: (0, 0)),
            pl.BlockSpec((br, n), lambda l, i: (i, 0)),
            pl.BlockSpec((1, F, F), lambda l, i: (l, 0, 0)),
            pl.BlockSpec((1, 1, F), lambda l, i: (l, 0, 0)),
            pl.BlockSpec((F, 256), lambda l, i: (0, 0)),
            pl.BlockSpec((1, 256), lambda l, i: (0, 0)),
            pl.BlockSpec((256, F), lambda l, i: (0, 0)),
            pl.BlockSpec((1, F), lambda l, i: (0, 0)),
            pl.BlockSpec((F, F), lambda l, i: (0, 0)),
            pl.BlockSpec((1, F), lambda l, i: (0, 0)),
        ],
        out_specs=pl.BlockSpec((1, F), lambda l, i: (0, 0)),
        out_shape=jax.ShapeDtypeStruct((1, F), jnp.float32),
        scratch_shapes=[
            pltpu.VMEM((n, F), jnp.bfloat16),
            pltpu.VMEM((n, F), jnp.bfloat16),
            pltpu.VMEM((n, F), jnp.float32),
            pltpu.VMEM((n, F), jnp.float32),
            pltpu.VMEM((1, F), jnp.float32),
        ],
        compiler_params=pltpu.CompilerParams(
            dimension_semantics=("arbitrary", "arbitrary")),
    )(nf_p, adj_bf, wg, bg.reshape(5, 1, F), Wd1, bd1.reshape(1, 256), Wd2,
      bd2.reshape(1, F), wd3_p, bd3_p)

    return out[0, :1]


# speedup vs baseline: 2.7681x; 1.3636x over previous
import functools

import jax
import jax.numpy as jnp
from jax.experimental import pallas as pl
from jax.experimental.pallas import tpu as pltpu

F = 128


def _gcn_body(nf_ref, a_ref, wg_ref, bg_ref, wd1_ref, bd1_ref, wd2_ref,
              bd2_ref, wd3_ref, bd3_ref, out_ref, zh_ref, zl_ref, h0_ref,
              h1_ref, pool_ref, *, br):
    l = pl.program_id(0)
    i = pl.program_id(1)
    nb = pl.num_programs(1)

    @pl.when(i == 0)
    def _():
        w = wg_ref[0]

        def _store_z(h):
            z = jnp.dot(h, w, preferred_element_type=jnp.float32,
                        precision=jax.lax.Precision.HIGHEST)
            zh = z.astype(jnp.bfloat16)
            zh_ref[...] = zh
            zl_ref[...] = (z - zh.astype(jnp.float32)).astype(jnp.bfloat16)

        @pl.when(l == 0)
        def _():
            _store_z(nf_ref[...])

        @pl.when(l % 2 == 1)
        def _():
            _store_z(h0_ref[...])

        @pl.when((l > 0) & (l % 2 == 0))
        def _():
            _store_z(h1_ref[...])

    a = a_ref[...]
    acc = jnp.dot(a, zh_ref[...], preferred_element_type=jnp.float32)
    h_out = jnp.maximum(acc + bg_ref[0], 0.0)

    @pl.when(l % 2 == 0)
    def _():
        h0_ref[pl.ds(i * br, br), :] = h_out

    @pl.when(l % 2 == 1)
    def _():
        h1_ref[pl.ds(i * br, br), :] = h_out

    @pl.when(l == 4)
    def _():
        @pl.when(i == 0)
        def _():
            pool_ref[...] = jnp.zeros_like(pool_ref)

        pool_ref[...] += jnp.sum(h_out, axis=0, keepdims=True)

        @pl.when(i == nb - 1)
        def _():
            p = pool_ref[...]
            nrm = jnp.sqrt(jnp.sum(p * p))
            x = p / jnp.maximum(nrm, 1e-12)
            x = jnp.maximum(
                jnp.dot(x, wd1_ref[...],
                        preferred_element_type=jnp.float32) + bd1_ref[...],
                0.0)
            x = jnp.maximum(
                jnp.dot(x, wd2_ref[...],
                        preferred_element_type=jnp.float32) + bd2_ref[...],
                0.0)
            out_ref[...] = (jnp.dot(x, wd3_ref[...],
                                    preferred_element_type=jnp.float32)
                            + bd3_ref[...])


def _pad2(w, r, c):
    return jnp.pad(w, ((0, r - w.shape[0]), (0, c - w.shape[1])))


def kernel(node_feats, adj, Wg1, bg1, Wg2, bg2, Wg3, bg3, Wg4, bg4, Wg5, bg5,
           Wd1, bd1, Wd2, bd2, Wd3, bd3):
    n = adj.shape[0]
    br = 400 if n % 400 == 0 else n // 4
    nb = n // br

    nf_p = jnp.pad(node_feats, ((0, 0), (0, F - node_feats.shape[1])))
    wg = jnp.stack([_pad2(w, F, F) for w in (Wg1, Wg2, Wg3, Wg4, Wg5)])
    bg = jnp.stack([jnp.pad(b, (0, F - b.shape[0])).reshape(1, F)
                    for b in (bg1, bg2, bg3, bg4, bg5)])
    wd3_p = _pad2(Wd3, F, F)
    bd3_p = jnp.pad(bd3.reshape(1, 1), ((0, 0), (0, F - 1)))

    adj_bf = adj.astype(jnp.bfloat16)

    out = pl.pallas_call(
        functools.partial(_gcn_body, br=br),
        grid=(5, nb),
        in_specs=[
            pl.BlockSpec((n, F), lambda l, i: (0, 0)),
            pl.BlockSpec((br, n), lambda l, i: (i, 0)),
            pl.BlockSpec((1, F, F), lambda l, i: (l, 0, 0)),
            pl.BlockSpec((1, 1, F), lambda l, i: (l, 0, 0)),
            pl.BlockSpec((F, 256), lambda l, i: (0, 0)),
            pl.BlockSpec((1, 256), lambda l, i: (0, 0)),
            pl.BlockSpec((256, F), lambda l, i: (0, 0)),
            pl.BlockSpec((1, F), lambda l, i: (0, 0)),
            pl.BlockSpec((F, F), lambda l, i: (0, 0)),
            pl.BlockSpec((1, F), lambda l, i: (0, 0)),
        ],
        out_specs=pl.BlockSpec((1, F), lambda l, i: (0, 0)),
        out_shape=jax.ShapeDtypeStruct((1, F), jnp.float32),
        scratch_shapes=[
            pltpu.VMEM((n, F), jnp.bfloat16),
            pltpu.VMEM((n, F), jnp.bfloat16),
            pltpu.VMEM((n, F), jnp.float32),
            pltpu.VMEM((n, F), jnp.float32),
            pltpu.VMEM((1, F), jnp.float32),
        ],
        compiler_params=pltpu.CompilerParams(
            dimension_semantics=("arbitrary", "arbitrary")),
    )(nf_p, adj_bf, wg, bg.reshape(5, 1, F), Wd1, bd1.reshape(1, 256), Wd2,
      bd2.reshape(1, F), wd3_p, bd3_p)

    return out[0, :1]
